# baseline (speedup 1.0000x reference)
import jax
import jax.numpy as jnp
from jax import lax
from jax.experimental import pallas as pl
from jax.experimental.pallas import tpu as pltpu

_B = 16384
_N = 1000
_CH = 512
_NCHUNK = _B // _CH
_NBUF = 4


def _loss_body(pred_hbm, lab_hbm, out_ref, buf, labbuf, sems, labsem):
    pltpu.make_async_copy(lab_hbm, labbuf, labsem).start()

    def _start(c):
        slot = lax.rem(c, _NBUF)
        pltpu.make_async_copy(
            pred_hbm.at[pl.ds(c * _CH, _CH), :], buf.at[slot], sems.at[slot]
        ).start()

    for k in range(_NBUF):
        _start(k)

    pltpu.make_async_copy(lab_hbm, labbuf, labsem).wait()

    def _step(c, acc):
        slot = lax.rem(c, _NBUF)
        pltpu.make_async_copy(
            pred_hbm.at[pl.ds(c * _CH, _CH), :], buf.at[slot], sems.at[slot]
        ).wait()
        p = buf[slot]
        lab = labbuf[pl.ds(c * _CH, _CH), :]
        col = lax.broadcasted_iota(jnp.int32, (_CH, _N), 1)
        q = jnp.where(col == lab, 1.0 - p, p)
        term = jnp.maximum(jnp.log(1.0 - q), -100.0)
        acc += jnp.sum(term)

        @pl.when(c + _NBUF < _NCHUNK)
        def _():
            _start(c + _NBUF)

        return acc

    acc = lax.fori_loop(0, _NCHUNK, _step, jnp.float32(0.0))
    out_ref[0, 0] = -acc / (_B * _N)


def kernel(pred, label):
    lab2 = label.astype(jnp.int32).reshape(_B, 1)
    out = pl.pallas_call(
        _loss_body,
        in_specs=[
            pl.BlockSpec(memory_space=pl.ANY),
            pl.BlockSpec(memory_space=pl.ANY),
        ],
        out_specs=pl.BlockSpec(memory_space=pltpu.SMEM),
        out_shape=jax.ShapeDtypeStruct((1, 1), jnp.float32),
        scratch_shapes=[
            pltpu.VMEM((_NBUF, _CH, _N), jnp.float32),
            pltpu.VMEM((_B, 1), jnp.int32),
            pltpu.SemaphoreType.DMA((_NBUF,)),
            pltpu.SemaphoreType.DMA,
        ],
    )(pred, lab2)
    return out[0, 0]
